# baseline (speedup 1.0000x reference)
import functools

import jax
import jax.numpy as jnp
from jax import lax
from jax.experimental import pallas as pl
from jax.experimental.pallas import tpu as pltpu
from jax.experimental.pallas import tpu_sc as plsc

_BATCH = 16384
_D = 32
_SUPER = 128
_V = 1_000_000
_SEG = 249_984
_TAIL = 4 * _SEG
_BLKW = 27776
_NI = _SEG // _BLKW + 1
_NSUP = _NI * _BLKW
_CHUNK = 128


def _relayout_body(in0, in1, in2, in3, e_ref, out_ref):
    i = pl.program_id(0)
    ins = (in0, in1, in2, in3)

    def place(x, u):
        return jax.lax.dot_general(
            x, e_ref[u], (((0,), (0,)), ((), ())),
            preferred_element_type=jnp.float32)

    @pl.when(i < _NI - 1)
    def _main():
        for w in range(_BLKW // _SUPER):
            sl = pl.ds(w * _SUPER, _SUPER)
            acc = place(in0[:, sl], 0)
            for u in range(1, 4):
                acc += place(ins[u][:, sl], u)
            out_ref[sl, :] = acc

    @pl.when(i == _NI - 1)
    def _tail():
        acc = place(in0[:, pl.ds(0, 16)], 0)
        for u in range(1, 4):
            acc += place(in0[:, pl.ds(16 * u, 16)], u)
        out_ref[pl.ds(0, 16), :] = acc


def _to_superrows(tT, eye4):
    nmain = _NI - 1
    tail_blk = _TAIL // _BLKW

    def imap(u):
        return lambda i: (0, jnp.where(i == nmain, tail_blk, u * nmain + i))

    return pl.pallas_call(
        _relayout_body,
        grid=(_NI,),
        in_specs=[pl.BlockSpec((_D, _BLKW), imap(u)) for u in range(4)]
        + [pl.BlockSpec(eye4.shape, lambda i: (0, 0, 0))],
        out_specs=pl.BlockSpec((_BLKW, _SUPER), lambda i: (i, 0)),
        out_shape=jax.ShapeDtypeStruct((_NSUP, _SUPER), jnp.float32),
    )(tT, tT, tT, tT, eye4)


@functools.cache
def _gather4():
    info = plsc.get_sparse_core_info()
    nw = info.num_cores * info.num_subcores
    b_per_w = _BATCH // nw
    n_ch = b_per_w // _CHUNK
    mesh = plsc.VectorSubcoreMesh(core_axis_name="c", subcore_axis_name="s")

    @functools.partial(
        pl.kernel,
        out_type=[jax.ShapeDtypeStruct((_BATCH, _SUPER), jnp.float32)] * 4,
        mesh=mesh,
        scratch_types=(
            [pltpu.VMEM((_CHUNK,), jnp.int32)] * 8
            + [pltpu.VMEM((_CHUNK, _SUPER), jnp.float32)] * 4
            + [pltpu.SemaphoreType.DMA] * 3
        ),
    )
    def gk(uidx_hbm, iidx_hbm, ug_hbm, ig_hbm, um_hbm, im_hbm,
           oug, oig, oum, oim,
           u0, u1, u2, u3, i0, i1, i2, i3,
           bug, big, bum, bim, gsem, wsem, ssem):
        wid = lax.axis_index("s") * info.num_cores + lax.axis_index("c")
        base = wid * b_per_w
        uvs = (u0, u1, u2, u3)
        ivs = (i0, i1, i2, i3)
        idx_cp = []
        for j in range(n_ch):
            sl = pl.ds(base + j * _CHUNK, _CHUNK)
            idx_cp.append(pltpu.async_copy(uidx_hbm.at[sl], uvs[j], ssem))
            idx_cp.append(pltpu.async_copy(iidx_hbm.at[sl], ivs[j], ssem))
        for c in idx_cp:
            c.wait()

        bufs = (bug, big, bum, bim)
        tabs = (ug_hbm, ig_hbm, um_hbm, im_hbm)
        outs = (oug, oig, oum, oim)

        w_descs = None
        for j in range(n_ch):
            idxs = (uvs[j], ivs[j], uvs[j], ivs[j])
            g_descs = [
                pltpu.async_copy(tabs[t].at[idxs[t]], bufs[t], gsem)
                for t in range(4)
            ]
            if w_descs is not None:
                for d in w_descs:
                    d.wait()
            for d in g_descs:
                d.wait()
            sl = pl.ds(base + j * _CHUNK, _CHUNK)
            w_descs = [
                pltpu.async_copy(bufs[t], outs[t].at[sl], wsem)
                for t in range(4)
            ]
        for d in w_descs:
            d.wait()

    return gk


def _dense_body(ug_ref, ig_ref, um_ref, im_ref, uw, iw,
                w0, b0, w1, b1, w2, b2, w3, b3, wpg, wph, bp, out_ref):
    def select(rows_ref, w_ref):
        m = w_ref[...][:, None]
        acc = jnp.zeros((rows_ref.shape[0], _D), jnp.float32)
        for mm in range(4):
            acc += jnp.where(m == mm,
                             rows_ref[:, mm * _D:(mm + 1) * _D], 0.0)
        return acc

    gmf = select(ug_ref, uw) * select(ig_ref, iw)
    h = jnp.concatenate([select(um_ref, uw), select(im_ref, iw)], axis=1)
    for w, b in ((w0, b0), (w1, b1), (w2, b2), (w3, b3)):
        h = jnp.maximum(
            jnp.dot(h, w[...], preferred_element_type=jnp.float32) + b[...], 0.0)
    pred = (jnp.dot(gmf, wpg[...], preferred_element_type=jnp.float32)
            + jnp.dot(h, wph[...], preferred_element_type=jnp.float32)
            + bp[...])
    out_ref[...] = jax.nn.sigmoid(pred)


def _dense(ug, ig, um, im, uw, iw,
           w0t, b0, w1t, b1, w2t, b2, w3t, b3, wpg, wph, bp):
    blk = 2048
    grid = (_BATCH // blk,)
    row = lambda i: (i, 0)
    vec = lambda i: (i,)
    fix = lambda i: (0, 0)
    fix1 = lambda i: (0,)
    in_specs = [
        pl.BlockSpec((blk, _SUPER), row),
        pl.BlockSpec((blk, _SUPER), row),
        pl.BlockSpec((blk, _SUPER), row),
        pl.BlockSpec((blk, _SUPER), row),
        pl.BlockSpec((blk,), vec),
        pl.BlockSpec((blk,), vec),
        pl.BlockSpec(w0t.shape, fix), pl.BlockSpec(b0.shape, fix1),
        pl.BlockSpec(w1t.shape, fix), pl.BlockSpec(b1.shape, fix1),
        pl.BlockSpec(w2t.shape, fix), pl.BlockSpec(b2.shape, fix1),
        pl.BlockSpec(w3t.shape, fix), pl.BlockSpec(b3.shape, fix1),
        pl.BlockSpec(wpg.shape, fix),
        pl.BlockSpec(wph.shape, fix),
        pl.BlockSpec(bp.shape, fix1),
    ]
    return pl.pallas_call(
        _dense_body,
        grid=grid,
        in_specs=in_specs,
        out_specs=pl.BlockSpec((blk, 1), row),
        out_shape=jax.ShapeDtypeStruct((_BATCH, 1), jnp.float32),
    )(ug, ig, um, im, uw, iw,
      w0t, b0, w1t, b1, w2t, b2, w3t, b3, wpg, wph, bp)


def _split_idx(idx):
    tail = idx >= _TAIL
    r = idx - _TAIL
    u = jnp.where(tail, r // 16, jnp.minimum(idx // _SEG, 3))
    q = jnp.where(tail, _SEG + r % 16, idx - u * _SEG)
    return q.astype(jnp.int32), u.astype(jnp.int32)


def kernel(user_indices, item_indices, ue_gmf, ie_gmf, ue_mlp, ie_mlp,
           W0, b0, W1, b1, W2, b2, W3, b3, Wp, bp):
    ui = user_indices.astype(jnp.int32)
    ii = item_indices.astype(jnp.int32)
    uq, uw = _split_idx(ui)
    iq, iw = _split_idx(ii)
    lanes = jnp.arange(_SUPER)[None, :]
    cols = jnp.arange(_D)[:, None]
    eye4 = jnp.stack(
        [(lanes == _D * u + cols).astype(jnp.float32) for u in range(4)])
    t = [_to_superrows(x.T, eye4) for x in (ue_gmf, ie_gmf, ue_mlp, ie_mlp)]
    ug, ig, umr, imr = _gather4()(uq, iq, *t)
    wpg = Wp[0, :_D].reshape(_D, 1)
    wph = Wp[0, _D:].reshape(-1, 1)
    return _dense(ug, ig, umr, imr, uw, iw,
                  W0.T, b0, W1.T, b1, W2.T, b2, W3.T, b3, wpg, wph, bp)


# speedup vs baseline: 1.0243x; 1.0243x over previous
import functools

import jax
import jax.numpy as jnp
from jax import lax
from jax.experimental import pallas as pl
from jax.experimental.pallas import tpu as pltpu
from jax.experimental.pallas import tpu_sc as plsc

_BATCH = 16384
_D = 32
_SUPER = 128
_V = 1_000_000
_SEG = 249_984
_TAIL = 4 * _SEG
_BLKW = 11904
_NI = _SEG // _BLKW + 1
_NSUP = _NI * _BLKW
_CHUNK = 128


def _relayout_body(in0, in1, in2, in3, e_ref, out_ref):
    i = pl.program_id(0)
    ins = (in0, in1, in2, in3)

    def place(x, u):
        return jax.lax.dot_general(
            x, e_ref[u], (((0,), (0,)), ((), ())),
            preferred_element_type=jnp.float32)

    @pl.when(i < _NI - 1)
    def _main():
        for w in range(_BLKW // _SUPER):
            sl = pl.ds(w * _SUPER, _SUPER)
            acc = place(in0[:, sl], 0)
            for u in range(1, 4):
                acc += place(ins[u][:, sl], u)
            out_ref[sl, :] = acc

    @pl.when(i == _NI - 1)
    def _tail():
        acc = place(in0[:, pl.ds(0, 16)], 0)
        for u in range(1, 4):
            acc += place(in0[:, pl.ds(16 * u, 16)], u)
        out_ref[pl.ds(0, 16), :] = acc


def _to_superrows(tT, eye4):
    nmain = _NI - 1
    tail_blk = _TAIL // _BLKW

    def imap(u):
        return lambda i: (0, jnp.where(i == nmain, tail_blk, u * nmain + i))

    return pl.pallas_call(
        _relayout_body,
        grid=(_NI,),
        in_specs=[pl.BlockSpec((_D, _BLKW), imap(u)) for u in range(4)]
        + [pl.BlockSpec(eye4.shape, lambda i: (0, 0, 0))],
        out_specs=pl.BlockSpec((_BLKW, _SUPER), lambda i: (i, 0)),
        out_shape=jax.ShapeDtypeStruct((_NSUP, _SUPER), jnp.float32),
    )(tT, tT, tT, tT, eye4)


@functools.cache
def _gather4():
    info = plsc.get_sparse_core_info()
    nw = info.num_cores * info.num_subcores
    b_per_w = _BATCH // nw
    n_ch = b_per_w // _CHUNK
    mesh = plsc.VectorSubcoreMesh(core_axis_name="c", subcore_axis_name="s")

    @functools.partial(
        pl.kernel,
        out_type=[jax.ShapeDtypeStruct((_BATCH, _SUPER), jnp.float32)] * 4,
        mesh=mesh,
        scratch_types=(
            [pltpu.VMEM((_CHUNK,), jnp.int32)] * 8
            + [pltpu.VMEM((_CHUNK, _SUPER), jnp.float32)] * 4
            + [pltpu.SemaphoreType.DMA] * 3
        ),
    )
    def gk(uidx_hbm, iidx_hbm, ug_hbm, ig_hbm, um_hbm, im_hbm,
           oug, oig, oum, oim,
           u0, u1, u2, u3, i0, i1, i2, i3,
           bug, big, bum, bim, gsem, wsem, ssem):
        wid = lax.axis_index("s") * info.num_cores + lax.axis_index("c")
        base = wid * b_per_w
        uvs = (u0, u1, u2, u3)
        ivs = (i0, i1, i2, i3)
        idx_cp = []
        for j in range(n_ch):
            sl = pl.ds(base + j * _CHUNK, _CHUNK)
            idx_cp.append(pltpu.async_copy(uidx_hbm.at[sl], uvs[j], ssem))
            idx_cp.append(pltpu.async_copy(iidx_hbm.at[sl], ivs[j], ssem))
        for c in idx_cp:
            c.wait()

        bufs = (bug, big, bum, bim)
        tabs = (ug_hbm, ig_hbm, um_hbm, im_hbm)
        outs = (oug, oig, oum, oim)

        w_descs = None
        for j in range(n_ch):
            idxs = (uvs[j], ivs[j], uvs[j], ivs[j])
            g_descs = [
                pltpu.async_copy(tabs[t].at[idxs[t]], bufs[t], gsem)
                for t in range(4)
            ]
            if w_descs is not None:
                for d in w_descs:
                    d.wait()
            for d in g_descs:
                d.wait()
            sl = pl.ds(base + j * _CHUNK, _CHUNK)
            w_descs = [
                pltpu.async_copy(bufs[t], outs[t].at[sl], wsem)
                for t in range(4)
            ]
        for d in w_descs:
            d.wait()

    return gk


def _dense_body(ug_ref, ig_ref, um_ref, im_ref, uw, iw,
                w0, b0, w1, b1, w2, b2, w3, b3, wpg, wph, bp, out_ref):
    def select(rows_ref, w_ref):
        m = w_ref[...][:, None]
        acc = jnp.zeros((rows_ref.shape[0], _D), jnp.float32)
        for mm in range(4):
            acc += jnp.where(m == mm,
                             rows_ref[:, mm * _D:(mm + 1) * _D], 0.0)
        return acc

    gmf = select(ug_ref, uw) * select(ig_ref, iw)
    h = jnp.concatenate([select(um_ref, uw), select(im_ref, iw)], axis=1)
    for w, b in ((w0, b0), (w1, b1), (w2, b2), (w3, b3)):
        h = jnp.maximum(
            jnp.dot(h, w[...], preferred_element_type=jnp.float32) + b[...], 0.0)
    pred = (jnp.dot(gmf, wpg[...], preferred_element_type=jnp.float32)
            + jnp.dot(h, wph[...], preferred_element_type=jnp.float32)
            + bp[...])
    out_ref[...] = jax.nn.sigmoid(pred)


def _dense(ug, ig, um, im, uw, iw,
           w0t, b0, w1t, b1, w2t, b2, w3t, b3, wpg, wph, bp):
    blk = 2048
    grid = (_BATCH // blk,)
    row = lambda i: (i, 0)
    vec = lambda i: (i,)
    fix = lambda i: (0, 0)
    fix1 = lambda i: (0,)
    in_specs = [
        pl.BlockSpec((blk, _SUPER), row),
        pl.BlockSpec((blk, _SUPER), row),
        pl.BlockSpec((blk, _SUPER), row),
        pl.BlockSpec((blk, _SUPER), row),
        pl.BlockSpec((blk,), vec),
        pl.BlockSpec((blk,), vec),
        pl.BlockSpec(w0t.shape, fix), pl.BlockSpec(b0.shape, fix1),
        pl.BlockSpec(w1t.shape, fix), pl.BlockSpec(b1.shape, fix1),
        pl.BlockSpec(w2t.shape, fix), pl.BlockSpec(b2.shape, fix1),
        pl.BlockSpec(w3t.shape, fix), pl.BlockSpec(b3.shape, fix1),
        pl.BlockSpec(wpg.shape, fix),
        pl.BlockSpec(wph.shape, fix),
        pl.BlockSpec(bp.shape, fix1),
    ]
    return pl.pallas_call(
        _dense_body,
        grid=grid,
        in_specs=in_specs,
        out_specs=pl.BlockSpec((blk, 1), row),
        out_shape=jax.ShapeDtypeStruct((_BATCH, 1), jnp.float32),
    )(ug, ig, um, im, uw, iw,
      w0t, b0, w1t, b1, w2t, b2, w3t, b3, wpg, wph, bp)


def _split_idx(idx):
    tail = idx >= _TAIL
    r = idx - _TAIL
    u = jnp.where(tail, r // 16, jnp.minimum(idx // _SEG, 3))
    q = jnp.where(tail, _SEG + r % 16, idx - u * _SEG)
    return q.astype(jnp.int32), u.astype(jnp.int32)


def kernel(user_indices, item_indices, ue_gmf, ie_gmf, ue_mlp, ie_mlp,
           W0, b0, W1, b1, W2, b2, W3, b3, Wp, bp):
    ui = user_indices.astype(jnp.int32)
    ii = item_indices.astype(jnp.int32)
    uq, uw = _split_idx(ui)
    iq, iw = _split_idx(ii)
    lanes = jnp.arange(_SUPER)[None, :]
    cols = jnp.arange(_D)[:, None]
    eye4 = jnp.stack(
        [(lanes == _D * u + cols).astype(jnp.float32) for u in range(4)])
    t = [_to_superrows(x.T, eye4) for x in (ue_gmf, ie_gmf, ue_mlp, ie_mlp)]
    ug, ig, umr, imr = _gather4()(uq, iq, *t)
    wpg = Wp[0, :_D].reshape(_D, 1)
    wph = Wp[0, _D:].reshape(-1, 1)
    return _dense(ug, ig, umr, imr, uw, iw,
                  W0.T, b0, W1.T, b1, W2.T, b2, W3.T, b3, wpg, wph, bp)
